# baseline (speedup 1.0000x reference)
import functools

import jax
import jax.numpy as jnp
from jax import lax
from jax.experimental import pallas as pl
from jax.experimental.pallas import tpu as pltpu
from jax.experimental.pallas import tpu_sc as plsc


def _make_transpose(V, D):
    NC, NS = 2, 16
    NW = NC * NS
    W = 128
    n_full = V // W
    tail = V - n_full * W
    per_w = n_full // NW
    n_extra = n_full - per_w * NW
    mesh = plsc.VectorSubcoreMesh(core_axis_name="c", subcore_axis_name="s",
                                  num_cores=NC, num_subcores=NS)

    @functools.partial(
        pl.kernel,
        out_type=jax.ShapeDtypeStruct((V // 2, 2 * D), jnp.float32),
        mesh=mesh,
        scratch_types=[
            pltpu.VMEM((D, W), jnp.float32),
            pltpu.VMEM((W // 2, 2 * D), jnp.float32),
            pltpu.VMEM((tail // 2, 2 * D), jnp.float32),
        ],
        name="wte_transpose",
        compiler_params=pltpu.CompilerParams(needs_layout_passes=False),
    )
    def transp(wt_hbm, tail_hbm, out_hbm, ib, ob, tb):
        wid = lax.axis_index("s") * NC + lax.axis_index("c")

        iota = lax.iota(jnp.int32, 16)
        row_idx = [iota + 16 * k for k in range(D // 16)]

        def do_chunk(chunk):
            col0 = chunk * W
            pltpu.sync_copy(wt_hbm.at[pl.ds(0, D), pl.ds(col0, W)], ib)

            def tr_body(r, c2):
                for j in range(2 * D // 16):
                    colv = jnp.full((16,), 2 * r + j // (D // 16), jnp.int32)
                    g = plsc.load_gather(ib, [row_idx[j % (D // 16)], colv])
                    ob[r, pl.ds(j * 16, 16)] = g
                return c2

            lax.fori_loop(0, W // 2, tr_body, 0, unroll=2)
            pltpu.sync_copy(ob, out_hbm.at[pl.ds(chunk * (W // 2), W // 2)])

        def chunk_loop(i, carry):
            do_chunk(wid + NW * i)
            return carry

        lax.fori_loop(0, per_w, chunk_loop, 0)

        @pl.when(wid < n_extra)
        def _():
            do_chunk(per_w * NW + wid)

        if tail:
            @pl.when(wid == n_extra)
            def _():
                pltpu.sync_copy(tail_hbm, tb)
                pltpu.sync_copy(
                    tb, out_hbm.at[pl.ds(n_full * (W // 2), tail // 2)])

    return transp


def _make_lookup(B, L, D, interpret=False):
    NC, NS = 2, 16
    NW = NC * NS
    assert B % NW == 0 and D == 64 and L == 200
    seq_per_w = B // NW
    n_chunks = seq_per_w // 2
    T = 2 * L
    GB = 80
    NG = T // GB
    R2 = T // 2
    mesh = plsc.VectorSubcoreMesh(core_axis_name="c", subcore_axis_name="s",
                                  num_cores=NC, num_subcores=NS)

    @functools.partial(
        pl.kernel,
        out_type=jax.ShapeDtypeStruct((B * L // 2, 2 * D), jnp.float32),
        mesh=mesh,
        scratch_types=[
            pltpu.VMEM((T,), jnp.int32),
            pltpu.VMEM((NG, GB), jnp.int32),
            pltpu.VMEM((T + 16,), jnp.int32),
            pltpu.VMEM((T, 2 * D), jnp.float32),
            pltpu.VMEM((R2, 2 * D), jnp.float32),
            pltpu.VMEM((L // 2, 2 * D), jnp.float32),
            pltpu.SemaphoreType.DMA,
        ],
        interpret=interpret,
        name="wte_wpe_lookup",
    )
    def lookup(ids_hbm, wte2_hbm, wpe2_hbm, out_hbm,
               idx_v, rows_v, off_v, big_v, out_v, wpe_v, sem):
        wid = lax.axis_index("s") * NC + lax.axis_index("c")

        pltpu.sync_copy(wpe2_hbm, wpe_v)

        def chunk_body(c, carry):
            s0 = wid * seq_per_w + 2 * c
            base_tok = s0 * L
            pltpu.sync_copy(ids_hbm.at[pl.ds(base_tok, T)], idx_v)

            def prep_body(j, c2):
                g = j // (GB // 16)
                jj = j % (GB // 16)
                v = idx_v[pl.ds(j * 16, 16)]
                rows_v[g, pl.ds(jj * 16, 16)] = lax.shift_right_logical(v, 1)
                off_v[pl.ds(j * 16, 16)] = (v & 1) * (2 * D // 2)
                return c2

            for j in range(T // 16):
                prep_body(j, 0)

            copies = []
            for g in range(NG):
                copies.append(pltpu.async_copy(
                    wte2_hbm.at[rows_v.at[g]],
                    big_v.at[pl.ds(g * GB, GB)], sem))
            for cp in copies:
                cp.wait()

            def add_body(r, c2):
                a = 2 * r
                ov = off_v[pl.ds(a, 16)]
                off_a = ov[0]
                off_b = ov[1]
                for j in range(D // 16):
                    sl_lo = pl.ds(j * 16, 16)
                    sl_hi = pl.ds(D + j * 16, 16)
                    out_v[r, sl_lo] = (big_v[a, pl.ds(off_a + j * 16, 16)]
                                       + wpe_v[r % (L // 2), sl_lo])
                    out_v[r, sl_hi] = (big_v[a + 1, pl.ds(off_b + j * 16, 16)]
                                       + wpe_v[r % (L // 2), sl_hi])
                return c2

            lax.fori_loop(0, R2, add_body, 0, unroll=2)

            pltpu.sync_copy(out_v, out_hbm.at[pl.ds(s0 * (L // 2), R2)])
            return carry

        lax.fori_loop(0, n_chunks, chunk_body, 0)

    return lookup


def kernel(input_ids, wte_table, wpe_table):
    B, L = input_ids.shape
    V, D = wte_table.shape
    ids_flat = input_ids.reshape(B * L).astype(jnp.int32)
    n_tail = V % 256
    tail_dense = wte_table[V - n_tail:].reshape(n_tail // 2, 2 * D)
    wte2 = _make_transpose(V, D)(wte_table.T, tail_dense)
    wpe2 = wpe_table[:L].reshape(L // 2, 2 * D)
    out2 = _make_lookup(B, L, D)(ids_flat, wte2, wpe2)
    return out2.reshape(B, L, D)


# speedup vs baseline: 2.5002x; 2.5002x over previous
import functools

import jax
import jax.numpy as jnp
from jax import lax
from jax.experimental import pallas as pl
from jax.experimental.pallas import tpu as pltpu
from jax.experimental.pallas import tpu_sc as plsc


def _make_lookup(B, L, V, D, interpret=False):
    NC, NS = 2, 16
    NW = NC * NS
    assert B % NW == 0 and L % 8 == 0
    seq_per_w = B // NW
    GB0 = 104
    GB1 = L - GB0
    mesh = plsc.VectorSubcoreMesh(core_axis_name="c", subcore_axis_name="s",
                                  num_cores=NC, num_subcores=NS)

    @functools.partial(
        pl.kernel,
        out_type=jax.ShapeDtypeStruct((B * L, D), jnp.float32),
        mesh=mesh,
        scratch_types=[
            pltpu.VMEM((L,), jnp.int32),
            pltpu.VMEM((L,), jnp.int32),
            pltpu.VMEM((L, D), jnp.float32),
            pltpu.VMEM((L, D), jnp.float32),
            pltpu.VMEM((L, D), jnp.float32),
            pltpu.SemaphoreType.DMA,
            pltpu.SemaphoreType.DMA,
            pltpu.SemaphoreType.DMA,
            pltpu.SemaphoreType.DMA,
        ],
        interpret=interpret,
        name="wte_wpe_lookup",
        compiler_params=pltpu.CompilerParams(use_tc_tiling_on_sc=False),
    )
    def lookup(ids_hbm, wte_hbm, wpe_hbm, out_hbm,
               idx_0, idx_1, rows_0, rows_1, wpe_v,
               gsem_0, gsem_1, osem_0, osem_1):
        wid = lax.axis_index("s") * NC + lax.axis_index("c")
        base_seq = wid * seq_per_w

        pltpu.sync_copy(wpe_hbm, wpe_v)

        idx_bufs = (idx_0, idx_1)
        row_bufs = (rows_0, rows_1)
        gsems = (gsem_0, gsem_1)
        osems = (osem_0, osem_1)

        def fetch(i, slot):
            base_tok = (base_seq + i) * L
            idx_v = idx_bufs[slot]
            rows_v = row_bufs[slot]
            pltpu.sync_copy(ids_hbm.at[pl.ds(base_tok, L)], idx_v)
            return [pltpu.async_copy(wte_hbm.at[idx_v.at[pl.ds(off, n)]],
                                     rows_v.at[pl.ds(off, n)],
                                     gsems[slot])
                    for off, n in ((0, GB0), (GB0, GB1))]

        def finish(i, slot, gathers):
            rows_v = row_bufs[slot]
            for cp in gathers:
                cp.wait()

            def add_body(r, c2):
                for j in range(D // 16):
                    sl = pl.ds(j * 16, 16)
                    rows_v[r, sl] = rows_v[r, sl] + wpe_v[r, sl]
                return c2

            lax.fori_loop(0, L, add_body, 0, unroll=4)
            base_tok = (base_seq + i) * L
            return pltpu.async_copy(rows_v, out_hbm.at[pl.ds(base_tok, L)],
                                    osems[slot])

        pending_g = {0: fetch(0, 0)}
        pending_o = {}
        for step in range(seq_per_w):
            slot = step % 2
            nxt = (step + 1) % 2
            if step + 1 < seq_per_w:
                if (step - 1) in pending_o:
                    pending_o.pop(step - 1).wait()
                pending_g[nxt] = fetch(step + 1, nxt)
            pending_o[step] = finish(step, slot, pending_g.pop(slot))
        for k in sorted(pending_o):
            pending_o.pop(k).wait()

    return lookup


def kernel(input_ids, wte_table, wpe_table):
    B, L = input_ids.shape
    V, D = wte_table.shape
    ids_flat = input_ids.reshape(B * L).astype(jnp.int32)
    wpe = wpe_table[:L]
    out = _make_lookup(B, L, V, D)(ids_flat, wte_table, wpe)
    return out.reshape(B, L, D)


# speedup vs baseline: 2.6708x; 1.0682x over previous
import functools

import jax
import jax.numpy as jnp
from jax import lax
from jax.experimental import pallas as pl
from jax.experimental.pallas import tpu as pltpu
from jax.experimental.pallas import tpu_sc as plsc


def _make_lookup(B, L, V, D, interpret=False):
    NC, NS = 2, 16
    NW = NC * NS
    assert B % NW == 0
    seq_per_w = B // NW
    mesh = plsc.VectorSubcoreMesh(core_axis_name="c", subcore_axis_name="s",
                                  num_cores=NC, num_subcores=NS)

    @functools.partial(
        pl.kernel,
        out_type=jax.ShapeDtypeStruct((B * L, D), jnp.float32),
        mesh=mesh,
        scratch_types=[
            pltpu.VMEM((L,), jnp.int32),
            pltpu.VMEM((L, D), jnp.float32),
            pltpu.VMEM((L, D), jnp.float32),
            pltpu.SemaphoreType.DMA,
        ],
        interpret=interpret,
        name="wte_wpe_lookup",
        compiler_params=pltpu.CompilerParams(use_tc_tiling_on_sc=False),
    )
    def lookup(ids_hbm, wte_hbm, wpe_hbm, out_hbm, idx_v, rows_v, wpe_v, sem):
        wid = lax.axis_index("s") * NC + lax.axis_index("c")

        pltpu.sync_copy(wpe_hbm, wpe_v)

        def seq_body(i, carry):
            base = (wid * seq_per_w + i) * L
            pltpu.sync_copy(ids_hbm.at[pl.ds(base, L)], idx_v)
            pltpu.async_copy(wte_hbm.at[idx_v], rows_v, sem).wait()

            def row_body(r, c2):
                for j in range(D // 16):
                    sl = pl.ds(j * 16, 16)
                    rows_v[r, sl] = rows_v[r, sl] + wpe_v[r, sl]
                return c2

            lax.fori_loop(0, L, row_body, 0)
            pltpu.sync_copy(rows_v, out_hbm.at[pl.ds(base, L)])
            return carry

        lax.fori_loop(0, seq_per_w, seq_body, 0)

    return lookup


def kernel(input_ids, wte_table, wpe_table):
    B, L = input_ids.shape
    V, D = wte_table.shape
    ids_flat = input_ids.reshape(B * L).astype(jnp.int32)
    wpe = wpe_table[:L]
    out = _make_lookup(B, L, V, D)(ids_flat, wte_table, wpe)
    return out.reshape(B, L, D)


# speedup vs baseline: 2.7854x; 1.0429x over previous
import functools

import jax
import jax.numpy as jnp
from jax import lax
from jax.experimental import pallas as pl
from jax.experimental.pallas import tpu as pltpu
from jax.experimental.pallas import tpu_sc as plsc


def _make_lookup(B, L, V, D, interpret=False):
    NC, NS = 2, 16
    NW = NC * NS
    assert B % NW == 0
    seq_per_w = B // NW
    mesh = plsc.VectorSubcoreMesh(core_axis_name="c", subcore_axis_name="s",
                                  num_cores=NC, num_subcores=NS)

    @functools.partial(
        pl.kernel,
        out_type=jax.ShapeDtypeStruct((B * L, D), jnp.float32),
        mesh=mesh,
        scratch_types=[
            pltpu.VMEM((L,), jnp.int32),
            pltpu.VMEM((L,), jnp.int32),
            pltpu.VMEM((L, D), jnp.float32),
            pltpu.VMEM((L, D), jnp.float32),
            pltpu.VMEM((L, D), jnp.float32),
            pltpu.SemaphoreType.DMA,
            pltpu.SemaphoreType.DMA,
        ],
        interpret=interpret,
        name="wte_wpe_lookup",
        compiler_params=pltpu.CompilerParams(use_tc_tiling_on_sc=False),
    )
    def lookup(ids_hbm, wte_hbm, wpe_hbm, out_hbm,
               idx_0, idx_1, rows_0, rows_1, wpe_v, sem_0, sem_1):
        wid = lax.axis_index("s") * NC + lax.axis_index("c")
        base_seq = wid * seq_per_w

        pltpu.sync_copy(wpe_hbm, wpe_v)

        idx_bufs = (idx_0, idx_1)
        row_bufs = (rows_0, rows_1)
        sems = (sem_0, sem_1)

        def fire(i, slot):
            base = (base_seq + i) * L
            pltpu.sync_copy(ids_hbm.at[pl.ds(base, L)], idx_bufs[slot])
            return pltpu.async_copy(wte_hbm.at[idx_bufs[slot]],
                                    row_bufs[slot], sems[slot])

        pending = fire(0, 0)
        for i in range(seq_per_w):
            slot = i % 2
            nxt = fire(i + 1, 1 - slot) if i + 1 < seq_per_w else None
            pending.wait()
            rows_v = row_bufs[slot]

            def row_body(r, c2):
                for j in range(D // 16):
                    sl = pl.ds(j * 16, 16)
                    rows_v[r, sl] = rows_v[r, sl] + wpe_v[r, sl]
                return c2

            lax.fori_loop(0, L, row_body, 0)
            base = (base_seq + i) * L
            pltpu.sync_copy(rows_v, out_hbm.at[pl.ds(base, L)])
            pending = nxt

    return lookup


def kernel(input_ids, wte_table, wpe_table):
    B, L = input_ids.shape
    V, D = wte_table.shape
    ids_flat = input_ids.reshape(B * L).astype(jnp.int32)
    wpe = wpe_table[:L]
    out = _make_lookup(B, L, V, D)(ids_flat, wte_table, wpe)
    return out.reshape(B, L, D)
